# baseline (speedup 1.0000x reference)
import functools

import jax
import jax.numpy as jnp
from jax import lax
from jax.experimental import pallas as pl
from jax.experimental.pallas import tpu as pltpu
from jax.experimental.pallas import tpu_sc as plsc

_P, _C = 96, 128
_ROWS_PER_WORKER = 16
_NUM_WORKERS = _P // _ROWS_PER_WORKER

_BLOCK_ROWS = 256


def _pe_gather(index, table):
    mesh = plsc.VectorSubcoreMesh(core_axis_name="c", subcore_axis_name="s")

    @functools.partial(
        pl.kernel,
        mesh=mesh,
        out_type=jax.ShapeDtypeStruct((_P, _C), jnp.float32),
        scratch_types=[
            pltpu.VMEM((_ROWS_PER_WORKER,), jnp.int32),
            pltpu.VMEM((_ROWS_PER_WORKER, _C), jnp.float32),
            pltpu.SemaphoreType.DMA,
        ],
    )
    def k(idx_hbm, table_hbm, out_hbm, idx_v, rows_v, sem):
        wid = lax.axis_index("s") * 2 + lax.axis_index("c")

        @pl.when(wid < _NUM_WORKERS)
        def _():
            base = wid * _ROWS_PER_WORKER
            pltpu.sync_copy(idx_hbm.at[pl.ds(base, _ROWS_PER_WORKER)], idx_v)
            pltpu.async_copy(table_hbm.at[idx_v], rows_v, sem).wait()
            pltpu.sync_copy(rows_v, out_hbm.at[pl.ds(base, _ROWS_PER_WORKER)])

    return k(index, table)


def _add_body(x_ref, pe_ref, o_ref):
    o_ref[...] = x_ref[...] + pe_ref[...]


def kernel(input_data, index, position_embedding):
    b, n, p, c = input_data.shape
    pe = _pe_gather(index.astype(jnp.int32), position_embedding)
    return (input_data.reshape(b * n, p, c) + pe[None]).reshape(b, n, p, c)

    bn = b * n
    x = input_data.reshape(bn, p, c)
    grid = (bn // _BLOCK_ROWS,)
    out = pl.pallas_call(
        _add_body,
        grid=grid,
        in_specs=[
            pl.BlockSpec((_BLOCK_ROWS, p, c), lambda i: (i, 0, 0)),
            pl.BlockSpec((p, c), lambda i: (0, 0)),
        ],
        out_specs=pl.BlockSpec((_BLOCK_ROWS, p, c), lambda i: (i, 0, 0)),
        out_shape=jax.ShapeDtypeStruct((bn, p, c), jnp.float32),
    )(x, pe)
    return out.reshape(b, n, p, c)


# speedup vs baseline: 1.1499x; 1.1499x over previous
import functools
import jax
import jax.numpy as jnp
from jax.experimental import pallas as pl
from jax.experimental.pallas import tpu as pltpu

_P, _C = 96, 128
_BLOCK_ROWS = 128


def _fused_body(idx_ref, x_ref, table_ref, o_ref, pe_ref):
    @pl.when(pl.program_id(0) == 0)
    def _():
        for j in range(_P):
            pe_ref[j, :] = table_ref[idx_ref[j], :]

    o_ref[...] = x_ref[...] + pe_ref[...]


def kernel(input_data, index, position_embedding):
    b, n, p, c = input_data.shape
    bn = b * n
    x = input_data.reshape(bn, p, c)
    grid_spec = pltpu.PrefetchScalarGridSpec(
        num_scalar_prefetch=1,
        grid=(bn // _BLOCK_ROWS,),
        in_specs=[
            pl.BlockSpec((_BLOCK_ROWS, p, c), lambda i, idx_ref: (i, 0, 0)),
            pl.BlockSpec((1000, c), lambda i, idx_ref: (0, 0)),
        ],
        out_specs=pl.BlockSpec((_BLOCK_ROWS, p, c), lambda i, idx_ref: (i, 0, 0)),
        scratch_shapes=[pltpu.VMEM((p, c), jnp.float32)],
    )
    out = pl.pallas_call(
        _fused_body,
        grid_spec=grid_spec,
        out_shape=jax.ShapeDtypeStruct((bn, p, c), jnp.float32),
    )(index.astype(jnp.int32), x, position_embedding)
    return out.reshape(b, n, p, c)
